# baseline (speedup 1.0000x reference)
import jax
import jax.numpy as jnp
from jax import lax
from jax.experimental import pallas as pl
from jax.experimental.pallas import tpu as pltpu
from jax.experimental.pallas import tpu_sc as plsc

B, N, M, D = 8, 1024, 48, 3
NUM_RS = 12
NUM_AT = 4
RC = 12.0
ETA = 4.0
ATOMS_PER_W = 256
NBLK = ATOMS_PER_W // 16

_CUT = (
    0.9999999945295114,
    -2.4674005624317474,
    2.029347420621804,
    -0.6675792150852533,
    0.11751490420139771,
    -0.012679491820734435,
    0.0007969553419935056,
)

_RSQRT_MAGIC = 0x5F3759DF


def _sc_body(x_hbm, nbrs_hbm, z_hbm, out_hbm, x_v, nbrs_v, z_v, stage_v, sem):
    c = lax.axis_index("c")
    s = lax.axis_index("s")
    wid = s * 2 + c
    b = wid // 4
    n0 = (wid % 4) * ATOMS_PER_W

    cp1 = pltpu.async_copy(x_hbm.at[b], x_v, sem)
    cp2 = pltpu.async_copy(nbrs_hbm.at[b, :, pl.ds(n0, ATOMS_PER_W)], nbrs_v, sem)
    cp3 = pltpu.async_copy(z_hbm.at[b, :, pl.ds(n0, ATOMS_PER_W)], z_v, sem)

    zeros = jnp.zeros((16,), jnp.float32)

    @plsc.parallel_loop(0, 5 * NUM_RS * ATOMS_PER_W // 16)
    def zero_body(i):
        stage_v[pl.ds(i * 16, 16)] = zeros

    cp1.wait()
    cp2.wait()
    cp3.wait()

    iota = lax.iota(jnp.int32, 16)

    def blk_body(blk, carry):
        atomv = blk * 16 + iota
        ci = (n0 + atomv) * 3
        cx = plsc.load_gather(x_v, [ci])
        cy = plsc.load_gather(x_v, [ci + 1])
        cz = plsc.load_gather(x_v, [ci + 2])

        @plsc.parallel_loop(0, M, unroll=4)
        def m_body(m):
            nbr = nbrs_v[m, pl.ds(blk * 16, 16)]
            zz = z_v[m, pl.ds(blk * 16, 16)]
            gi = nbr * 3
            gx = plsc.load_gather(x_v, [gi])
            gy = plsc.load_gather(x_v, [gi + 1])
            gz = plsc.load_gather(x_v, [gi + 2])
            dx = gx - cx
            dy = gy - cy
            dz = gz - cz
            r2 = jnp.maximum(dx * dx + dy * dy + dz * dz, 1e-12)
            yi = _RSQRT_MAGIC - (plsc.bitcast(r2, jnp.int32) >> 1)
            y = plsc.bitcast(yi, jnp.float32)
            h = -0.5 * r2
            y = y * (1.5 + h * y * y)
            y = y * (1.5 + h * y * y)
            r = r2 * y
            v = jnp.minimum(r2 * (1.0 / (RC * RC)), 1.0)
            fc = jnp.float32(_CUT[6])
            fc = fc * v + _CUT[5]
            fc = fc * v + _CUT[4]
            fc = fc * v + _CUT[3]
            fc = fc * v + _CUT[2]
            fc = fc * v + _CUT[1]
            fc = fc * v + _CUT[0]
            a0 = r2 * (-ETA)
            b8 = r * (2.0 * ETA)
            slot = jnp.where(zz >= 6, zz - 5, 4)
            slot = jnp.where(zz == 1, 0, slot)
            base = slot * (NUM_RS * ATOMS_PER_W) + atomv
            for rs in range(NUM_RS):
                e = jnp.exp(b8 * jnp.float32(rs) + (a0 - jnp.float32(ETA * rs * rs)))
                plsc.addupdate_scatter(stage_v, [base + rs * ATOMS_PER_W], e * fc)

        return carry

    lax.fori_loop(0, NBLK, blk_body, 0)

    obase = b * N + n0
    handles = []
    for rs in range(NUM_RS):
        for a in range(NUM_AT):
            src = stage_v.at[pl.ds((a * NUM_RS + rs) * ATOMS_PER_W, ATOMS_PER_W)]
            dst = out_hbm.at[rs * NUM_AT + a, pl.ds(obase, ATOMS_PER_W)]
            handles.append(pltpu.async_copy(src, dst, sem))
    for hh in handles:
        hh.wait()


def kernel(X, Nbrs, Nbrs_Z):
    x_flat = X.reshape(B, N * D)
    nbrs_t = Nbrs.transpose(0, 2, 1)
    z_t = Nbrs_Z.transpose(0, 2, 1)
    mesh = plsc.VectorSubcoreMesh(core_axis_name="c", subcore_axis_name="s")
    out = pl.kernel(
        _sc_body,
        out_type=jax.ShapeDtypeStruct((NUM_RS * NUM_AT, B * N), jnp.float32),
        mesh=mesh,
        compiler_params=pltpu.CompilerParams(needs_layout_passes=False),
        scratch_types=[
            pltpu.VMEM((N * D,), jnp.float32),
            pltpu.VMEM((M, ATOMS_PER_W), jnp.int32),
            pltpu.VMEM((M, ATOMS_PER_W), jnp.int32),
            pltpu.VMEM((5 * NUM_RS * ATOMS_PER_W,), jnp.float32),
            pltpu.SemaphoreType.DMA,
        ],
    )(x_flat, nbrs_t, z_t)
    return out.reshape(NUM_RS * NUM_AT, B, N)


# speedup vs baseline: 1.6223x; 1.6223x over previous
import jax
import jax.numpy as jnp
from jax import lax
from jax.experimental import pallas as pl
from jax.experimental.pallas import tpu as pltpu
from jax.experimental.pallas import tpu_sc as plsc

B, N, M, D = 8, 1024, 48, 3
NUM_RS = 12
NUM_AT = 4
RC = 12.0
ETA = 4.0
ATOMS_PER_W = 256
NBLK = ATOMS_PER_W // 16

_CUT = (
    0.9999999945295114,
    -2.4674005624317474,
    2.029347420621804,
    -0.6675792150852533,
    0.11751490420139771,
    -0.012679491820734435,
    0.0007969553419935056,
)

_RSQRT_MAGIC = 0x5F3759DF


def _sc_body(x_hbm, nbrs_hbm, z_hbm, out_hbm, x_v, nbrs_v, z_v, stage_v, sem):
    c = lax.axis_index("c")
    s = lax.axis_index("s")
    wid = s * 2 + c
    b = wid // 4
    n0 = (wid % 4) * ATOMS_PER_W

    cp1 = pltpu.async_copy(x_hbm.at[b], x_v, sem)
    cp2 = pltpu.async_copy(nbrs_hbm.at[b, :, pl.ds(n0, ATOMS_PER_W)], nbrs_v, sem)
    cp3 = pltpu.async_copy(z_hbm.at[b, :, pl.ds(n0, ATOMS_PER_W)], z_v, sem)

    zeros = jnp.zeros((16,), jnp.float32)

    @plsc.parallel_loop(0, 5 * NUM_RS * ATOMS_PER_W // 16)
    def zero_body(i):
        stage_v[pl.ds(i * 16, 16)] = zeros

    cp1.wait()
    cp2.wait()
    cp3.wait()

    iota = lax.iota(jnp.int32, 16)

    if True:
        @plsc.parallel_loop(0, NBLK * M, unroll=2)
        def m_body(q):
            blk = q & (NBLK - 1)
            m = q >> 4
            atomv = blk * 16 + iota
            ci = (n0 + atomv) * 3
            cx = plsc.load_gather(x_v, [ci])
            cy = plsc.load_gather(x_v, [ci + 1])
            cz = plsc.load_gather(x_v, [ci + 2])
            nbr = nbrs_v[m, pl.ds(blk * 16, 16)]
            zz = z_v[m, pl.ds(blk * 16, 16)]
            gi = nbr * 3
            gx = plsc.load_gather(x_v, [gi])
            gy = plsc.load_gather(x_v, [gi + 1])
            gz = plsc.load_gather(x_v, [gi + 2])
            dx = gx - cx
            dy = gy - cy
            dz = gz - cz
            r2 = jnp.maximum(dx * dx + dy * dy + dz * dz, 1e-12)
            yi = _RSQRT_MAGIC - (plsc.bitcast(r2, jnp.int32) >> 1)
            y = plsc.bitcast(yi, jnp.float32)
            h = -0.5 * r2
            y = y * (1.5 + h * y * y)
            y = y * (1.5 + h * y * y)
            r = r2 * y
            v = jnp.minimum(r2 * (1.0 / (RC * RC)), 1.0)
            fc = jnp.float32(_CUT[6])
            fc = fc * v + _CUT[5]
            fc = fc * v + _CUT[4]
            fc = fc * v + _CUT[3]
            fc = fc * v + _CUT[2]
            fc = fc * v + _CUT[1]
            fc = fc * v + _CUT[0]
            a0 = r2 * (-ETA)
            b8 = r * (2.0 * ETA)
            slot = jnp.where(zz >= 6, zz - 5, 4)
            slot = jnp.where(zz == 1, 0, slot)
            base = slot * (NUM_RS * ATOMS_PER_W) + atomv
            for rs in range(NUM_RS):
                e = jnp.exp(b8 * jnp.float32(rs) + (a0 - jnp.float32(ETA * rs * rs)))
                plsc.addupdate_scatter(stage_v, [base + rs * ATOMS_PER_W], e * fc)

    obase = b * N + n0
    handles = []
    for rs in range(NUM_RS):
        for a in range(NUM_AT):
            src = stage_v.at[pl.ds((a * NUM_RS + rs) * ATOMS_PER_W, ATOMS_PER_W)]
            dst = out_hbm.at[rs * NUM_AT + a, pl.ds(obase, ATOMS_PER_W)]
            handles.append(pltpu.async_copy(src, dst, sem))
    for hh in handles:
        hh.wait()


def kernel(X, Nbrs, Nbrs_Z):
    x_flat = X.reshape(B, N * D)
    nbrs_t = Nbrs.transpose(0, 2, 1)
    z_t = Nbrs_Z.transpose(0, 2, 1)
    mesh = plsc.VectorSubcoreMesh(core_axis_name="c", subcore_axis_name="s")
    out = pl.kernel(
        _sc_body,
        out_type=jax.ShapeDtypeStruct((NUM_RS * NUM_AT, B * N), jnp.float32),
        mesh=mesh,
        compiler_params=pltpu.CompilerParams(needs_layout_passes=False),
        scratch_types=[
            pltpu.VMEM((N * D,), jnp.float32),
            pltpu.VMEM((M, ATOMS_PER_W), jnp.int32),
            pltpu.VMEM((M, ATOMS_PER_W), jnp.int32),
            pltpu.VMEM((5 * NUM_RS * ATOMS_PER_W,), jnp.float32),
            pltpu.SemaphoreType.DMA,
        ],
    )(x_flat, nbrs_t, z_t)
    return out.reshape(NUM_RS * NUM_AT, B, N)
